# baseline (speedup 1.0000x reference)
import functools

import jax
import jax.numpy as jnp
from jax import lax
from jax.experimental import pallas as pl
from jax.experimental.pallas import tpu as pltpu
from jax.experimental.pallas import tpu_sc as plsc

BATCH = 16384
EMB_K = 64

_info = plsc.get_sparse_core_info()
_NC, _NS = _info.num_cores, _info.num_subcores
_NW = _NC * _NS
_BPW = BATCH // _NW


def _sc_gather_body(uidx_hbm, vidx_hbm, w_hbm, h_hbm, u_out, v_out,
                    uidx_v, vidx_v, urows_v, vrows_v, sem_u, sem_v):
    wid = lax.axis_index("s") * _NC + lax.axis_index("c")
    base = wid * _BPW
    pltpu.sync_copy(uidx_hbm.at[pl.ds(base, _BPW)], uidx_v)
    pltpu.sync_copy(vidx_hbm.at[pl.ds(base, _BPW)], vidx_v)
    cu = pltpu.async_copy(w_hbm.at[uidx_v], urows_v, sem_u)
    cv = pltpu.async_copy(h_hbm.at[vidx_v], vrows_v, sem_v)
    cu.wait()
    cv.wait()
    pltpu.sync_copy(urows_v, u_out.at[pl.ds(base, _BPW)])
    pltpu.sync_copy(vrows_v, v_out.at[pl.ds(base, _BPW)])


@jax.jit
def _sc_gather(user_idx, item_idx, w_table, h_table):
    mesh = plsc.VectorSubcoreMesh(core_axis_name="c", subcore_axis_name="s")
    f = functools.partial(
        pl.kernel,
        mesh=mesh,
        out_type=[
            jax.ShapeDtypeStruct((BATCH, EMB_K), jnp.float32),
            jax.ShapeDtypeStruct((BATCH, EMB_K), jnp.float32),
        ],
        scratch_types=[
            pltpu.VMEM((_BPW,), jnp.int32),
            pltpu.VMEM((_BPW,), jnp.int32),
            pltpu.VMEM((_BPW, EMB_K), jnp.float32),
            pltpu.VMEM((_BPW, EMB_K), jnp.float32),
            pltpu.SemaphoreType.DMA,
            pltpu.SemaphoreType.DMA,
        ],
        compiler_params=pltpu.CompilerParams(use_tc_tiling_on_sc=False),
    )(_sc_gather_body)
    return f(user_idx, item_idx, w_table, h_table)


def _tc_mlp_body(u_ref, v_ref, a_ref, b_ref, bias_ref, w2_ref, o_ref):
    h = lax.dot_general(u_ref[...], a_ref[...], (((1,), (1,)), ((), ())),
                        preferred_element_type=jnp.float32)
    h = h + lax.dot_general(v_ref[...], b_ref[...], (((1,), (1,)), ((), ())),
                            preferred_element_type=jnp.float32)
    h = h + bias_ref[...]
    h = jnp.maximum(h, 0.0)
    o_ref[...] = lax.dot_general(h, w2_ref[...], (((1,), (0,)), ((), ())),
                                 preferred_element_type=jnp.float32)


_TC_BLK = 2048


@jax.jit
def _tc_mlp(u_emb, v_emb, a_w, b_w, bias, w2_col):
    grid = (BATCH // _TC_BLK,)
    return pl.pallas_call(
        _tc_mlp_body,
        grid=grid,
        in_specs=[
            pl.BlockSpec((_TC_BLK, EMB_K), lambda i: (i, 0)),
            pl.BlockSpec((_TC_BLK, EMB_K), lambda i: (i, 0)),
            pl.BlockSpec((EMB_K, EMB_K), lambda i: (0, 0)),
            pl.BlockSpec((EMB_K, EMB_K), lambda i: (0, 0)),
            pl.BlockSpec((1, EMB_K), lambda i: (0, 0)),
            pl.BlockSpec((EMB_K, 1), lambda i: (0, 0)),
        ],
        out_specs=pl.BlockSpec((_TC_BLK, 1), lambda i: (i, 0)),
        out_shape=jax.ShapeDtypeStruct((BATCH, 1), jnp.float32),
    )(u_emb, v_emb, a_w, b_w, bias, w2_col)


def kernel(x, W_table, H_table, linear1_w, linear1_b, linear2_w):
    user_idx = x[:, 0].astype(jnp.int32)
    item_idx = x[:, 1].astype(jnp.int32)
    u_emb, v_emb = _sc_gather(user_idx, item_idx, W_table, H_table)
    a_w = linear1_w[:, :EMB_K]
    b_w = linear1_w[:, EMB_K:]
    bias = linear1_b.reshape(1, EMB_K)
    w2_col = linear2_w.reshape(EMB_K, 1)
    return _tc_mlp(u_emb, v_emb, a_w, b_w, bias, w2_col)
